# baseline (speedup 1.0000x reference)
import jax
import jax.numpy as jnp
from jax import lax
from jax.experimental import pallas as pl
from jax.experimental.pallas import tpu as pltpu
from jax.experimental.pallas import tpu_sc as plsc
import functools

NUM_EMOTIONS = 1000
SEQ = 32
HID = 768
BATCH = 1024

NC = 2
NS = 16
NW = NC * NS

SC_CHUNKS = 4
SC_W = SEQ // SC_CHUNKS
BG = NW // SC_CHUNKS
B_PER_G = BATCH // BG
ROWS_PER_STREAM = 8
N_STREAMS = B_PER_G // ROWS_PER_STREAM

MASK_PER_W = BATCH // NW


def _mesh_kernel():
    mesh = plsc.VectorSubcoreMesh(core_axis_name="c", subcore_axis_name="s")

    @functools.partial(
        pl.kernel,
        mesh=mesh,
        out_type=[
            jax.ShapeDtypeStruct((BATCH, SEQ, HID), jnp.float32),
            jax.ShapeDtypeStruct((BATCH, SEQ), jnp.int32),
        ],
        scratch_types=[
            pltpu.VMEM((B_PER_G,), jnp.int32),
            pltpu.VMEM((MASK_PER_W,), jnp.int32),
            pltpu.VMEM((MASK_PER_W, SEQ), jnp.int32),
            pltpu.VMEM((ROWS_PER_STREAM, SC_W, HID), jnp.float32),
            pltpu.VMEM((ROWS_PER_STREAM, SC_W, HID), jnp.float32),
            pltpu.SemaphoreType.DMA,
            pltpu.SemaphoreType.DMA,
            pltpu.SemaphoreType.DMA,
        ],
    )
    def body(table_hbm, ids_hbm, mask_hbm, cond_out, mask_out,
             ids_v, mids_v, mrows_v, buf_a, buf_b, sem_a, sem_b, msem):
        wid = lax.axis_index("c") * NS + lax.axis_index("s")
        bg = wid // SC_CHUNKS
        dc = wid % SC_CHUNKS

        mbase = wid * MASK_PER_W
        pltpu.sync_copy(ids_hbm.at[pl.ds(mbase, MASK_PER_W)], mids_v)
        handles = []
        for blk in range(MASK_PER_W // 16):
            vec = mids_v[pl.ds(blk * 16, 16)]
            for j in range(16):
                i = blk * 16 + j
                rid = vec[j]
                handles.append(pltpu.async_copy(
                    mask_hbm.at[rid], mrows_v.at[i], msem))

        base = bg * B_PER_G
        pltpu.sync_copy(ids_hbm.at[pl.ds(base, B_PER_G)], ids_v)

        bufs = (buf_a, buf_b)
        sems = (sem_a, sem_b)

        def gather(g, b):
            return pltpu.async_copy(
                table_hbm.at[ids_v.at[pl.ds(g * ROWS_PER_STREAM, ROWS_PER_STREAM)],
                             pl.ds(dc * SC_W, SC_W), :],
                bufs[b], sems[b])

        hs = [gather(0, 0), None]
        for g in range(N_STREAMS):
            cur = g % 2
            if g + 1 < N_STREAMS:
                hs[1 - cur] = gather(g + 1, 1 - cur)
            hs[cur].wait()
            pltpu.sync_copy(
                bufs[cur],
                cond_out.at[pl.ds(base + g * ROWS_PER_STREAM, ROWS_PER_STREAM),
                            pl.ds(dc * SC_W, SC_W), :])

        for h in handles:
            h.wait()
        pltpu.sync_copy(mrows_v, mask_out.at[pl.ds(mbase, MASK_PER_W), :])

    return body


def kernel(emotion_ids, conditioning, attention_masks):
    ids = emotion_ids.astype(jnp.int32)
    cond_out, mask_out = _mesh_kernel()(conditioning, ids, attention_masks)
    return cond_out, mask_out


# speedup vs baseline: 1.0031x; 1.0031x over previous
import jax
import jax.numpy as jnp
from jax import lax
from jax.experimental import pallas as pl
from jax.experimental.pallas import tpu as pltpu
from jax.experimental.pallas import tpu_sc as plsc
import functools

NUM_EMOTIONS = 1000
SEQ = 32
HID = 768
BATCH = 1024

NC = 2
NS = 16
NW = NC * NS

SC_CHUNKS = 4
SC_W = SEQ // SC_CHUNKS
BG = NW // SC_CHUNKS
B_PER_G = BATCH // BG
ROWS_PER_STREAM = 8
N_STREAMS = B_PER_G // ROWS_PER_STREAM

MASK_PER_W = BATCH // NW


def _mesh_kernel():
    mesh = plsc.VectorSubcoreMesh(core_axis_name="c", subcore_axis_name="s")

    @functools.partial(
        pl.kernel,
        mesh=mesh,
        out_type=[
            jax.ShapeDtypeStruct((BATCH, SEQ, HID), jnp.float32),
            jax.ShapeDtypeStruct((BATCH, SEQ), jnp.int32),
        ],
        scratch_types=[
            pltpu.VMEM((B_PER_G,), jnp.int32),
            pltpu.VMEM((MASK_PER_W,), jnp.int32),
            pltpu.VMEM((MASK_PER_W, SEQ), jnp.int32),
            pltpu.VMEM((ROWS_PER_STREAM, SC_W, HID), jnp.float32),
            pltpu.VMEM((ROWS_PER_STREAM, SC_W, HID), jnp.float32),
            pltpu.SemaphoreType.DMA,
            pltpu.SemaphoreType.DMA,
            pltpu.SemaphoreType.DMA,
        ],
    )
    def body(table_hbm, ids_hbm, mask_hbm, cond_out, mask_out,
             ids_v, mids_v, mrows_v, buf_a, buf_b, sem_a, sem_b, msem):
        wid = lax.axis_index("c") * NS + lax.axis_index("s")
        bg = wid // SC_CHUNKS
        dc = wid % SC_CHUNKS

        base = bg * B_PER_G
        pltpu.sync_copy(ids_hbm.at[pl.ds(base, B_PER_G)], ids_v)

        bufs = (buf_a, buf_b)
        sems = (sem_a, sem_b)

        def gather(g, b):
            return pltpu.async_copy(
                table_hbm.at[ids_v.at[pl.ds(g * ROWS_PER_STREAM, ROWS_PER_STREAM)],
                             pl.ds(dc * SC_W, SC_W), :],
                bufs[b], sems[b])

        hs = [gather(0, 0), gather(1, 1)]

        mbase = wid * MASK_PER_W
        pltpu.sync_copy(ids_hbm.at[pl.ds(mbase, MASK_PER_W)], mids_v)
        handles = []
        for blk in range(MASK_PER_W // 16):
            vec = mids_v[pl.ds(blk * 16, 16)]
            for j in range(16):
                i = blk * 16 + j
                rid = vec[j]
                handles.append(pltpu.async_copy(
                    mask_hbm.at[rid], mrows_v.at[i], msem))

        for g in range(N_STREAMS):
            cur = g % 2
            hs[cur].wait()
            pltpu.sync_copy(
                bufs[cur],
                cond_out.at[pl.ds(base + g * ROWS_PER_STREAM, ROWS_PER_STREAM),
                            pl.ds(dc * SC_W, SC_W), :])
            if g + 2 < N_STREAMS:
                hs[cur] = gather(g + 2, cur)

        for h in handles:
            h.wait()
        pltpu.sync_copy(mrows_v, mask_out.at[pl.ds(mbase, MASK_PER_W), :])

    return body


def kernel(emotion_ids, conditioning, attention_masks):
    ids = emotion_ids.astype(jnp.int32)
    cond_out, mask_out = _mesh_kernel()(conditioning, ids, attention_masks)
    return cond_out, mask_out
